# baseline (speedup 1.0000x reference)
import functools

import jax
import jax.numpy as jnp
from jax import lax
from jax.experimental import pallas as pl
from jax.experimental.pallas import tpu as pltpu
from jax.experimental.pallas import tpu_sc as plsc

_BM = 4096
_V = 1024
_D = 32
_N = 65536

_NC = 2
_NS = 16
_NW = _NC * _NS
_BPW = _N // _NW
_G = _BPW // 16


def _assign_body(x_ref, ncta_ref, selt_ref, codes_ref):
    x = x_ref[...]
    nct = ncta_ref[:_D, :]
    csq = ncta_ref[_D:_D + 1, :]
    selt = selt_ref[...]
    s = jax.lax.dot_general(x, nct, (((1,), (0,)), ((), ())),
                            preferred_element_type=jnp.float32)
    a = s + csq
    amin = jnp.min(a, axis=1, keepdims=True)
    oh = jnp.where(a == amin, 1.0, 0.0)
    ci = jax.lax.dot_general(selt, oh, (((1,), (1,)), ((), ())),
                             preferred_element_type=jnp.float32)
    cnt = ci[0:1, :]
    lo = jnp.floor(ci[1:2, :] / cnt)
    hi = jnp.floor(ci[2:3, :] / cnt)
    codes_ref[...] = (hi * 32.0 + lo).astype(jnp.int32)[None]


def _gather_kernel(codes_hbm, table_hbm, out_hbm, idx_v, tab_v, rows_v, sem):
    wid = lax.axis_index("s") * _NC + lax.axis_index("c")
    pltpu.sync_copy(table_hbm, tab_v)
    pltpu.sync_copy(codes_hbm.at[wid], idx_v)

    lane = lax.iota(jnp.int32, 16)

    @plsc.parallel_loop(0, _G, unroll=8)
    def _(g):
        cvec = idx_v[pl.ds(g * 16, 16)]
        r = lax.shift_right_logical(cvec, 2)
        c0 = (cvec & 3) * 32
        m = g * 16 + lane
        orow = lax.shift_right_logical(m, 2)
        oc0 = (m & 3) * 32
        for d in range(_D):
            vals = plsc.load_gather(tab_v, [r, c0 + d])
            plsc.store_scatter(rows_v, [orow, oc0 + d], vals)

    pltpu.sync_copy(rows_v, out_hbm.at[wid])


@jax.jit
def kernel(z, codebook):
    B, T, D = z.shape
    flat = z.reshape(-1, D)
    grid = _N // _BM

    csq = jnp.sum(codebook * codebook, axis=1)[None, :]
    ncta = jnp.concatenate([-2.0 * codebook.T, csq], axis=0)
    iota = lax.iota(jnp.float32, _V)
    selt = jnp.concatenate([
        jnp.ones((1, _V), jnp.float32),
        (jnp.mod(iota, 32.0))[None, :],
        jnp.floor(iota / 32.0)[None, :],
        jnp.zeros((5, _V), jnp.float32),
    ], axis=0)

    codes = pl.pallas_call(
        _assign_body,
        grid=(grid,),
        in_specs=[
            pl.BlockSpec((_BM, D), lambda i: (i, 0)),
            pl.BlockSpec((D + 1, _V), lambda i: (0, 0)),
            pl.BlockSpec((8, _V), lambda i: (0, 0)),
        ],
        out_specs=pl.BlockSpec((1, 1, _BM), lambda i: (i, 0, 0)),
        out_shape=jax.ShapeDtypeStruct((grid, 1, _BM), jnp.int32),
    )(flat, ncta, selt)

    codes_w = codes.reshape(_NW, _BPW)
    table128 = codebook.reshape(_V * _D // 128, 128)

    gather = functools.partial(
        pl.kernel,
        mesh=plsc.VectorSubcoreMesh(core_axis_name="c", subcore_axis_name="s"),
        out_type=jax.ShapeDtypeStruct((_NW, _BPW * _D // 128, 128), jnp.float32),
        scratch_types=[
            pltpu.VMEM((_BPW,), jnp.int32),
            pltpu.VMEM((_V * _D // 128, 128), jnp.float32),
            pltpu.VMEM((_BPW * _D // 128, 128), jnp.float32),
            pltpu.SemaphoreType.DMA,
        ],
        compiler_params=pltpu.CompilerParams(use_tc_tiling_on_sc=True,
                                             needs_layout_passes=False),
    )(_gather_kernel)

    out = gather(codes_w, table128)
    return out.reshape(B, T, D)


# speedup vs baseline: 1.2347x; 1.2347x over previous
import functools

import jax
import jax.numpy as jnp
from jax import lax
from jax.experimental import pallas as pl
from jax.experimental.pallas import tpu as pltpu
from jax.experimental.pallas import tpu_sc as plsc

_BM = 4096
_V = 1024
_D = 32
_N = 65536

_NC = 2
_NS = 16
_NW = _NC * _NS
_BPW = _N // _NW
_IDXC = 128
_NIDX = _BPW // _IDXC


def _assign_body(x_ref, ncta_ref, selt_ref, codes_ref):
    x = x_ref[...]
    nct = ncta_ref[:_D, :]
    csq = ncta_ref[_D:_D + 1, :]
    selt = selt_ref[...]
    s = jax.lax.dot_general(x, nct, (((1,), (0,)), ((), ())),
                            preferred_element_type=jnp.float32)
    a = s + csq
    amin = jnp.min(a, axis=1, keepdims=True)
    oh = jnp.where(a == amin, 1.0, 0.0)
    ci = jax.lax.dot_general(selt, oh, (((1,), (1,)), ((), ())),
                             preferred_element_type=jnp.float32)
    cnt = ci[0:1, :]
    lo = jnp.floor(ci[1:2, :] / cnt)
    hi = jnp.floor(ci[2:3, :] / cnt)
    codes_ref[...] = (hi * 32.0 + lo).astype(jnp.int32)[None]


def _gather_kernel(codes_hbm, table_hbm, out_hbm, idx_v, rows_v, sem):
    wid = lax.axis_index("s") * _NC + lax.axis_index("c")
    pltpu.sync_copy(codes_hbm.at[wid], idx_v)
    copies = []
    for j in range(_NIDX):
        copies.append(pltpu.async_copy(
            table_hbm.at[idx_v.at[j]],
            rows_v.at[pl.ds(j * _IDXC, _IDXC)],
            sem))
    for cp in copies:
        cp.wait()
    pltpu.sync_copy(rows_v, out_hbm.at[wid])


@jax.jit
def kernel(z, codebook):
    B, T, D = z.shape
    flat = z.reshape(-1, D)
    grid = _N // _BM

    csq = jnp.sum(codebook * codebook, axis=1)[None, :]
    ncta = jnp.concatenate([-2.0 * codebook.T, csq], axis=0)
    iota = lax.iota(jnp.float32, _V)
    selt = jnp.concatenate([
        jnp.ones((1, _V), jnp.float32),
        (jnp.mod(iota, 32.0))[None, :],
        jnp.floor(iota / 32.0)[None, :],
        jnp.zeros((5, _V), jnp.float32),
    ], axis=0)

    codes = pl.pallas_call(
        _assign_body,
        grid=(grid,),
        in_specs=[
            pl.BlockSpec((_BM, D), lambda i: (i, 0)),
            pl.BlockSpec((D + 1, _V), lambda i: (0, 0)),
            pl.BlockSpec((8, _V), lambda i: (0, 0)),
        ],
        out_specs=pl.BlockSpec((1, 1, _BM), lambda i: (i, 0, 0)),
        out_shape=jax.ShapeDtypeStruct((grid, 1, _BM), jnp.int32),
    )(flat, ncta, selt)

    codes_w = codes.reshape(_NW, _NIDX, _IDXC)

    gather = functools.partial(
        pl.kernel,
        mesh=plsc.VectorSubcoreMesh(core_axis_name="c", subcore_axis_name="s"),
        out_type=jax.ShapeDtypeStruct((_NW, _BPW, _D), jnp.float32),
        scratch_types=[
            pltpu.VMEM((_NIDX, _IDXC), jnp.int32),
            pltpu.VMEM((_BPW, _D), jnp.float32),
            pltpu.SemaphoreType.DMA,
        ],
        compiler_params=pltpu.CompilerParams(use_tc_tiling_on_sc=False),
    )(_gather_kernel)

    out = gather(codes_w, codebook)
    return out.reshape(B, T, D)
